# baseline (speedup 1.0000x reference)
import functools

import jax
import jax.numpy as jnp
from jax import lax
from jax.experimental import pallas as pl
from jax.experimental.pallas import tpu as pltpu

_TILE_V = 4096


def _body(stok_ref, order_ref, starts_ref, tbl_ref, w_ref, b_ref, o_ref,
          x_ref, tpose_ref, sem, *, tile_v):
    p = pl.program_id(0)
    j = pl.program_id(1)

    @pl.when(p == 0)
    def _gather():
        tpose_ref[...] = jnp.transpose(tbl_ref[...])
        lo = j * tile_v

        def issue_one(i, carry):
            local = stok_ref[i] - lo
            dst = order_ref[i]
            pltpu.make_async_copy(
                tpose_ref.at[pl.ds(local, 1), :],
                x_ref.at[pl.ds(dst, 1), :],
                sem,
            ).start()
            return carry

        def drain_one(i, carry):
            pltpu.make_async_copy(
                tpose_ref.at[pl.ds(0, 1), :],
                x_ref.at[pl.ds(0, 1), :],
                sem,
            ).wait()
            return carry

        lax.fori_loop(starts_ref[j], starts_ref[j + 1], issue_one, 0)
        lax.fori_loop(starts_ref[j], starts_ref[j + 1], drain_one, 0)

    @pl.when(p == 1)
    def _matmul():
        acc = lax.dot_general(
            w_ref[...],
            x_ref[...],
            (((0,), (1,)), ((), ())),
            preferred_element_type=jnp.float32,
        )
        o_ref[...] = acc + jnp.transpose(b_ref[...])


def kernel(input_token, emb_table, fc_weight, fc_bias):
    V, D = emb_table.shape
    B = input_token.shape[0]
    tile_v = _TILE_V
    grid_j = pl.cdiv(V, tile_v)

    tokens = input_token.astype(jnp.int32)
    order = jnp.argsort(tokens).astype(jnp.int32)
    sorted_tok = jnp.take(tokens, order)
    shift = tile_v.bit_length() - 1
    counts = jnp.zeros(grid_j, jnp.int32).at[tokens >> shift].add(1)
    starts = jnp.concatenate(
        [jnp.zeros(1, jnp.int32), jnp.cumsum(counts, dtype=jnp.int32)]
    )

    table_t = emb_table.T
    w_t = fc_weight.T
    bias2d = fc_bias.reshape(1, V)

    grid_spec = pltpu.PrefetchScalarGridSpec(
        num_scalar_prefetch=3,
        grid=(2, grid_j),
        in_specs=[
            pl.BlockSpec(
                (D, tile_v), lambda p, j, *_: (0, jnp.where(p == 0, j, 0))
            ),
            pl.BlockSpec(
                (D, tile_v), lambda p, j, *_: (0, jnp.where(p == 1, j, 0))
            ),
            pl.BlockSpec(
                (1, tile_v), lambda p, j, *_: (0, jnp.where(p == 1, j, 0))
            ),
        ],
        out_specs=pl.BlockSpec(
            (tile_v, B), lambda p, j, *_: (jnp.where(p == 1, j, 0), 0)
        ),
        scratch_shapes=[
            pltpu.VMEM((B, D), jnp.float32),
            pltpu.VMEM((tile_v, D), jnp.float32),
            pltpu.SemaphoreType.DMA,
        ],
    )
    out_t = pl.pallas_call(
        functools.partial(_body, tile_v=tile_v),
        grid_spec=grid_spec,
        out_shape=jax.ShapeDtypeStruct((V, B), jnp.float32),
        compiler_params=pltpu.CompilerParams(
            dimension_semantics=("arbitrary", "arbitrary"),
        ),
    )(sorted_tok, order, starts, table_t, w_t, bias2d)
    return out_t.T
